# baseline (speedup 1.0000x reference)
import functools

import jax
import jax.numpy as jnp
from jax.experimental import pallas as pl
from jax.experimental.pallas import tpu as pltpu

S, E, C = 4096, 8, 1024
TILE_S = 512


def _fused_kernel(logits_ref, m1_ref, m2_ref, loc1_ref, loc2_ref,
                  combine_ref, laux_ref, acc_ref):
    i = pl.program_id(0)
    n = pl.num_programs(0)

    lg = logits_ref[...]
    m1 = m1_ref[...]
    m2 = m2_ref[...]

    mx = jnp.max(lg, axis=1, keepdims=True)
    ex = jnp.exp(lg - mx)
    gates = ex / jnp.sum(ex, axis=1, keepdims=True)

    @pl.when(i == 0)
    def _():
        acc_ref[...] = jnp.zeros_like(acc_ref)

    acc_ref[0:1, :] += jnp.sum(gates, axis=0, keepdims=True)
    acc_ref[1:2, :] += jnp.sum(m1, axis=0, keepdims=True)

    g1s = jnp.sum(gates * m1, axis=1, keepdims=True)
    g2s = jnp.sum(gates * m2, axis=1, keepdims=True)
    denom = jnp.maximum(g1s + g2s, jnp.finfo(jnp.float32).eps)
    g1 = (g1s / denom) * m1
    g2 = (g2s / denom) * m2

    loc1 = loc1_ref[...]
    loc2 = loc2_ref[...]
    combine_ref[...] = (g1[:, :, None] * loc1[:, None, :]
                        + g2[:, :, None] * loc2[:, None, :])

    @pl.when(i == n - 1)
    def _():
        me_ce = acc_ref[0:1, :] * acc_ref[1:2, :]
        scale = jnp.float32(E) / jnp.float32(S * S)
        laux_ref[...] = jnp.sum(me_ce, axis=1, keepdims=True) * scale


@functools.partial(jax.jit, static_argnames=("interpret",))
def kernel(logits, mask1_float, mask2_float, locations1_sc, locations2_sc,
           interpret=False):
    grid = (S // TILE_S,)
    laux, combine = pl.pallas_call(
        lambda *refs: _fused_kernel(*refs[:5], refs[6], refs[5], refs[7]),
        grid=grid,
        in_specs=[
            pl.BlockSpec((TILE_S, E), lambda i: (i, 0)),
            pl.BlockSpec((TILE_S, E), lambda i: (i, 0)),
            pl.BlockSpec((TILE_S, E), lambda i: (i, 0)),
            pl.BlockSpec((TILE_S, C), lambda i: (i, 0)),
            pl.BlockSpec((TILE_S, C), lambda i: (i, 0)),
        ],
        out_specs=[
            pl.BlockSpec((1, 1), lambda i: (0, 0)),
            pl.BlockSpec((TILE_S, E, C), lambda i: (i, 0, 0)),
        ],
        out_shape=[
            jax.ShapeDtypeStruct((1, 1), jnp.float32),
            jax.ShapeDtypeStruct((S, E, C), jnp.float32),
        ],
        scratch_shapes=[pltpu.VMEM((2, E), jnp.float32)],
        compiler_params=pltpu.CompilerParams(
            dimension_semantics=("arbitrary",),
        ),
        interpret=interpret,
    )(logits, mask1_float, mask2_float, locations1_sc, locations2_sc)
    return laux[0, 0], combine


# speedup vs baseline: 1.0016x; 1.0016x over previous
import functools

import jax
import jax.numpy as jnp
from jax.experimental import pallas as pl
from jax.experimental.pallas import tpu as pltpu

S, E, C = 4096, 8, 1024
TILE_S = 512


def _fused_kernel(logits_ref, m1_ref, m2_ref, loc1_ref, loc2_ref,
                  laux_ref, combine_ref, acc_ref):
    i = pl.program_id(0)
    n = pl.num_programs(0)

    lg = logits_ref[...]
    m1 = m1_ref[...]
    m2 = m2_ref[...]

    mx = jnp.max(lg, axis=1, keepdims=True)
    ex = jnp.exp(lg - mx)
    gates = ex / jnp.sum(ex, axis=1, keepdims=True)

    @pl.when(i == 0)
    def _():
        acc_ref[...] = jnp.zeros_like(acc_ref)

    acc_ref[0:1, :] += jnp.sum(gates, axis=0, keepdims=True)
    acc_ref[1:2, :] += jnp.sum(m1, axis=0, keepdims=True)

    g1s = jnp.sum(gates * m1, axis=1, keepdims=True)
    g2s = jnp.sum(gates * m2, axis=1, keepdims=True)
    denom = jnp.maximum(g1s + g2s, jnp.finfo(jnp.float32).eps)
    g1 = (g1s / denom) * m1
    g2 = (g2s / denom) * m2

    loc1 = loc1_ref[...]
    loc2 = loc2_ref[...]
    out = g1[:, :, None] * loc1[:, None, :] + g2[:, :, None] * loc2[:, None, :]
    combine_ref[...] = out

    @pl.when(i == n - 1)
    def _():
        me_ce = acc_ref[0:1, :] * acc_ref[1:2, :]
        scale = jnp.float32(E) / jnp.float32(S * S)
        laux_ref[...] = jnp.sum(me_ce, axis=1, keepdims=True) * scale


@functools.partial(jax.jit, static_argnames=("interpret",))
def kernel(logits, mask1_float, mask2_float, locations1_sc, locations2_sc,
           interpret=False):
    grid = (S // TILE_S,)
    laux, combine = pl.pallas_call(
        _fused_kernel,
        grid=grid,
        in_specs=[
            pl.BlockSpec((TILE_S, E), lambda i: (i, 0)),
            pl.BlockSpec((TILE_S, E), lambda i: (i, 0)),
            pl.BlockSpec((TILE_S, E), lambda i: (i, 0)),
            pl.BlockSpec((TILE_S, C), lambda i: (i, 0)),
            pl.BlockSpec((TILE_S, C), lambda i: (i, 0)),
        ],
        out_specs=[
            pl.BlockSpec((1, 1), lambda i: (0, 0)),
            pl.BlockSpec((TILE_S, E, C), lambda i: (i, 0, 0)),
        ],
        out_shape=[
            jax.ShapeDtypeStruct((1, 1), jnp.float32),
            jax.ShapeDtypeStruct((S, E, C), jnp.float32),
        ],
        scratch_shapes=[pltpu.VMEM((2, E), jnp.float32)],
        compiler_params=pltpu.CompilerParams(
            dimension_semantics=("arbitrary",),
        ),
        interpret=interpret,
    )(logits, mask1_float, mask2_float, locations1_sc, locations2_sc)
    return laux[0, 0], combine
